# baseline (speedup 1.0000x reference)
import functools

import jax
import jax.numpy as jnp
from jax.experimental import pallas as pl
from jax.experimental.pallas import tpu as pltpu

LANE = 128
TM = 256
NBUF = 4
VMEM_LIMIT = 64 * 1024 * 1024


def _round_up(x, m):
    return (x + m - 1) // m * m


def _pad2d(x, rows, cols):
    if x.shape == (rows, cols):
        return x
    return jnp.pad(x, ((0, rows - x.shape[0]), (0, cols - x.shape[1])))


def _fused_kernel(x_ref, w1_ref, a_hbm, b1_ref, w2_ref, b2_ref, o_ref,
                  z1_scr, acc_scr, abuf, asem, *, tm, t):
    def dma_in(slot, step):
        pltpu.make_async_copy(
            a_hbm.at[pl.ds(step * tm, tm)], abuf.at[slot],
            asem.at[slot]).start()

    def wait_in(slot):
        pltpu.make_async_copy(
            a_hbm.at[pl.ds(0, tm)], abuf.at[slot], asem.at[slot]).wait()

    dma_in(0, 0)

    @pl.when(t > 1)
    def _():
        dma_in(1, 1)

    @pl.when(t > 2)
    def _():
        dma_in(2, 2)

    xb = x_ref[...].astype(jnp.bfloat16)
    w1 = w1_ref[...].astype(jnp.bfloat16)
    z1_scr[...] = jnp.dot(
        xb, w1, preferred_element_type=jnp.float32).astype(jnp.bfloat16)
    acc_scr[...] = jnp.broadcast_to(
        jnp.swapaxes(b2_ref[...], 0, 1), acc_scr.shape)

    w2 = w2_ref[...].astype(jnp.bfloat16)
    b1 = b1_ref[...]

    def body(step, carry):
        nxt = jax.lax.rem(step + 3, NBUF)

        @pl.when(step + 3 < t)
        def _():
            dma_in(nxt, step + 3)

        cur = jax.lax.rem(step, NBUF)
        wait_in(cur)

        ab = abuf[cur].astype(jnp.bfloat16)
        acc1 = jnp.dot(ab, z1_scr[...], preferred_element_type=jnp.float32)
        h = jnp.maximum(acc1 + b1, 0.0)
        z2_k = jnp.dot(h.astype(jnp.bfloat16), w2,
                       preferred_element_type=jnp.float32).astype(jnp.bfloat16)

        z2t = jnp.swapaxes(z2_k, 0, 1)
        acc_scr[...] += jnp.dot(z2t, ab, preferred_element_type=jnp.float32)
        return carry

    jax.lax.fori_loop(0, t, body, 0)

    logits = acc_scr[...]
    m = jnp.max(logits, axis=0, keepdims=True)
    s = logits - m
    lse = jnp.log(jnp.sum(jnp.exp(s), axis=0, keepdims=True))
    o_ref[...] = jnp.swapaxes(s - lse, 0, 1).astype(o_ref.dtype)


def kernel(x, a_hat, w1, b1, w2, b2):
    n, f = x.shape
    n_cls = w2.shape[1]
    tm = TM
    np_ = _round_up(n, tm)
    fp = _round_up(f, LANE)
    t = np_ // tm

    a_p = _pad2d(a_hat, np_, np_)
    x_p = _pad2d(x, np_, fp)
    w1_p = _pad2d(w1, fp, LANE)
    b1_p = b1.reshape(1, -1)
    b2_p = b2.reshape(1, -1)

    out = pl.pallas_call(
        functools.partial(_fused_kernel, tm=tm, t=t),
        out_shape=jax.ShapeDtypeStruct((np_, n_cls), jnp.float32),
        grid=(1,),
        in_specs=[
            pl.BlockSpec((np_, fp), lambda i: (0, 0)),
            pl.BlockSpec((fp, LANE), lambda i: (0, 0)),
            pl.BlockSpec(memory_space=pl.ANY),
            pl.BlockSpec((1, LANE), lambda i: (0, 0)),
            pl.BlockSpec((LANE, n_cls), lambda i: (0, 0)),
            pl.BlockSpec((1, n_cls), lambda i: (0, 0)),
        ],
        out_specs=pl.BlockSpec((np_, n_cls), lambda i: (0, 0)),
        scratch_shapes=[
            pltpu.VMEM((np_, LANE), jnp.bfloat16),
            pltpu.VMEM((n_cls, np_), jnp.float32),
            pltpu.VMEM((NBUF, tm, np_), jnp.float32),
            pltpu.SemaphoreType.DMA((NBUF,)),
        ],
        compiler_params=pltpu.CompilerParams(
            dimension_semantics=("arbitrary",),
            vmem_limit_bytes=VMEM_LIMIT),
    )(x_p, w1_p, a_p, b1_p, w2, b2_p)
    return out[:n]


# speedup vs baseline: 1.0119x; 1.0119x over previous
import functools

import jax
import jax.numpy as jnp
from jax.experimental import pallas as pl
from jax.experimental.pallas import tpu as pltpu

LANE = 128
TM = 256
NBUF = 3
VMEM_LIMIT = 64 * 1024 * 1024


def _round_up(x, m):
    return (x + m - 1) // m * m


def _pad2d(x, rows, cols):
    if x.shape == (rows, cols):
        return x
    return jnp.pad(x, ((0, rows - x.shape[0]), (0, cols - x.shape[1])))


def _fused_kernel(x_ref, w1_ref, a_hbm, b1_ref, w2_ref, b2_ref, o_ref,
                  z1_scr, acc_scr, abuf, asem, *, tm, t):
    def dma_in(slot, step):
        pltpu.make_async_copy(
            a_hbm.at[pl.ds(step * tm, tm)], abuf.at[slot],
            asem.at[slot]).start()

    def wait_in(slot):
        pltpu.make_async_copy(
            a_hbm.at[pl.ds(0, tm)], abuf.at[slot], asem.at[slot]).wait()

    dma_in(0, 0)

    @pl.when(t > 1)
    def _():
        dma_in(1, 1)

    xb = x_ref[...].astype(jnp.bfloat16)
    w1 = w1_ref[...].astype(jnp.bfloat16)
    z1_scr[...] = jnp.dot(
        xb, w1, preferred_element_type=jnp.float32).astype(jnp.bfloat16)
    acc_scr[...] = jnp.broadcast_to(
        jnp.swapaxes(b2_ref[...], 0, 1), acc_scr.shape)

    w2 = w2_ref[...].astype(jnp.bfloat16)
    b1 = b1_ref[...]

    def body(step, carry):
        nxt = jax.lax.rem(step + 2, NBUF)

        @pl.when(step + 2 < t)
        def _():
            dma_in(nxt, step + 2)

        cur = jax.lax.rem(step, NBUF)
        wait_in(cur)

        ab = abuf[cur].astype(jnp.bfloat16)
        acc1 = jnp.dot(ab, z1_scr[...], preferred_element_type=jnp.float32)
        h = jnp.maximum(acc1 + b1, 0.0)
        z2_k = jnp.dot(h.astype(jnp.bfloat16), w2,
                       preferred_element_type=jnp.float32).astype(jnp.bfloat16)

        z2t = jnp.swapaxes(z2_k, 0, 1)
        acc_scr[...] += jnp.dot(z2t, ab, preferred_element_type=jnp.float32)
        return carry

    jax.lax.fori_loop(0, t, body, 0)

    logits = acc_scr[...]
    m = jnp.max(logits, axis=0, keepdims=True)
    s = logits - m
    lse = jnp.log(jnp.sum(jnp.exp(s), axis=0, keepdims=True))
    o_ref[...] = jnp.swapaxes(s - lse, 0, 1).astype(o_ref.dtype)


def kernel(x, a_hat, w1, b1, w2, b2):
    n, f = x.shape
    n_cls = w2.shape[1]
    tm = TM
    np_ = _round_up(n, tm)
    fp = _round_up(f, LANE)
    t = np_ // tm

    a_p = _pad2d(a_hat, np_, np_)
    x_p = _pad2d(x, np_, fp)
    w1_p = _pad2d(w1, fp, LANE)
    b1_p = b1.reshape(1, -1)
    b2_p = b2.reshape(1, -1)

    out = pl.pallas_call(
        functools.partial(_fused_kernel, tm=tm, t=t),
        out_shape=jax.ShapeDtypeStruct((np_, n_cls), jnp.float32),
        grid=(1,),
        in_specs=[
            pl.BlockSpec((np_, fp), lambda i: (0, 0)),
            pl.BlockSpec((fp, LANE), lambda i: (0, 0)),
            pl.BlockSpec(memory_space=pl.ANY),
            pl.BlockSpec((1, LANE), lambda i: (0, 0)),
            pl.BlockSpec((LANE, n_cls), lambda i: (0, 0)),
            pl.BlockSpec((1, n_cls), lambda i: (0, 0)),
        ],
        out_specs=pl.BlockSpec((np_, n_cls), lambda i: (0, 0)),
        scratch_shapes=[
            pltpu.VMEM((np_, LANE), jnp.bfloat16),
            pltpu.VMEM((n_cls, np_), jnp.float32),
            pltpu.VMEM((NBUF, tm, np_), jnp.float32),
            pltpu.SemaphoreType.DMA((NBUF,)),
        ],
        compiler_params=pltpu.CompilerParams(
            dimension_semantics=("arbitrary",),
            vmem_limit_bytes=VMEM_LIMIT),
    )(x_p, w1_p, a_p, b1_p, w2, b2_p)
    return out[:n]
